# baseline (speedup 1.0000x reference)
import functools

import jax
import jax.numpy as jnp
from jax import lax
from jax.experimental import pallas as pl
from jax.experimental.pallas import tpu as pltpu
from jax.experimental.pallas import tpu_sc as plsc

_D = 64
_LANES = 16

_NC = 2
_NS = 16
_NW = _NC * _NS

_SCHUNK = 128

_STRIPE = 62464

_CK = 65536


def _make_count(n_tokens: int, vocab: int):
    per_w = n_tokens // _NW
    nch = per_w // _SCHUNK
    last_stripe = vocab - (_NS - 1) * _STRIPE

    mesh = plsc.VectorSubcoreMesh(core_axis_name="c", subcore_axis_name="s")

    @functools.partial(
        pl.kernel,
        out_type=jax.ShapeDtypeStruct((_NC, vocab), jnp.float32),
        mesh=mesh,
        scratch_types=[
            pltpu.VMEM((nch, _SCHUNK), jnp.int32),
            pltpu.VMEM((_SCHUNK,), jnp.float32),
            pltpu.VMEM((last_stripe,), jnp.float32),
            pltpu.VMEM_SHARED((vocab,), jnp.float32),
            pltpu.SemaphoreType.DMA,
            pltpu.SemaphoreType.DMA,
        ],
    )
    def count(idx_hbm, out_hbm, idx_v, ones_v, zero_v, cnt_sh, sem_i, sem_z):
        cid = lax.axis_index("c")
        sid = lax.axis_index("s")
        wid = sid * _NC + cid

        ci = pltpu.async_copy(idx_hbm.at[wid], idx_v, sem_i)

        for g in range(_SCHUNK // _LANES):
            ones_v[pl.ds(g * _LANES, _LANES)] = jnp.ones(
                (_LANES,), jnp.float32
            )

        zeros16 = jnp.zeros((_LANES,), jnp.float32)

        def zbody(i, _):
            zero_v[pl.ds(i * _LANES, _LANES)] = zeros16
            return 0

        lax.fori_loop(0, last_stripe // _LANES, zbody, 0)

        base = sid * _STRIPE

        @pl.when(sid == _NS - 1)
        def _():
            pltpu.async_copy(
                zero_v, cnt_sh.at[pl.ds(base, last_stripe)], sem_z
            ).wait()

        @pl.when(sid != _NS - 1)
        def _():
            pltpu.async_copy(
                zero_v.at[pl.ds(0, _STRIPE)],
                cnt_sh.at[pl.ds(base, _STRIPE)],
                sem_z,
            ).wait()

        ci.wait()
        plsc.subcore_barrier()

        for j in range(nch):
            pltpu.sync_copy(ones_v, cnt_sh.at[idx_v.at[j]], add=True)

        plsc.subcore_barrier()

        @pl.when(sid == 0)
        def _():
            pltpu.sync_copy(cnt_sh, out_hbm.at[cid])

    return count


def _matvec_head_kernel(nblk, vocab, cnt_ref, t_ref, w_ref, b_ref, o_ref,
                        acc_ref):
    k = pl.program_id(0)
    csum = cnt_ref[0:1, :] + cnt_ref[1:2, :]
    t = t_ref[...]

    def tail_psum(_):
        gid = (nblk - 1) * _CK + lax.broadcasted_iota(
            jnp.int32, (1, _CK), 1
        )
        prod = jnp.where(gid < vocab, t * csum, 0.0)
        return jnp.sum(prod, axis=1, keepdims=True)

    def main_psum(_):
        return jnp.sum(t * csum, axis=1, keepdims=True)

    psum = lax.cond(k == nblk - 1, tail_psum, main_psum, 0)

    @pl.when(k == 0)
    def _():
        acc_ref[...] = jnp.zeros_like(acc_ref)

    acc_ref[...] += psum

    @pl.when(k == nblk - 1)
    def _():
        bow = acc_ref[...]
        logits = (
            jnp.dot(w_ref[...], bow, preferred_element_type=jnp.float32)
            + b_ref[...]
        )
        m = jnp.max(logits, axis=0, keepdims=True)
        e = jnp.exp(logits - m)
        lse = jnp.log(jnp.sum(e, axis=0, keepdims=True)) + m
        o_ref[...] = logits - lse


def kernel(inputs, emb_table, W, b):
    n_tokens = inputs.shape[0]
    vocab = emb_table.shape[0]
    num_labels = W.shape[0]

    per_w = n_tokens // _NW
    idx = inputs.astype(jnp.int32).reshape(_NW, per_w // _SCHUNK, _SCHUNK)

    cnt2 = _make_count(n_tokens, vocab)(idx)

    table_t = emb_table.T

    nblk = (vocab + _CK - 1) // _CK
    out_col = pl.pallas_call(
        functools.partial(_matvec_head_kernel, nblk, vocab),
        grid=(nblk,),
        in_specs=[
            pl.BlockSpec((_NC, _CK), lambda k: (0, k)),
            pl.BlockSpec((_D, _CK), lambda k: (0, k)),
            pl.BlockSpec((num_labels, _D), lambda k: (0, 0)),
            pl.BlockSpec((num_labels, 1), lambda k: (0, 0)),
        ],
        out_specs=pl.BlockSpec((num_labels, 1), lambda k: (0, 0)),
        out_shape=jax.ShapeDtypeStruct((num_labels, 1), jnp.float32),
        scratch_shapes=[pltpu.VMEM((_D, 1), jnp.float32)],
    )(cnt2, table_t, W, b.reshape(num_labels, 1))

    return out_col.reshape(1, num_labels)


# speedup vs baseline: 1.1513x; 1.1513x over previous
import functools

import jax
import jax.numpy as jnp
from jax import lax
from jax.experimental import pallas as pl
from jax.experimental.pallas import tpu as pltpu
from jax.experimental.pallas import tpu_sc as plsc

_D = 64
_LANES = 16

_NC = 2
_NS = 16
_NW = _NC * _NS

_SCHUNK = 128

_STRIPE = 62464

_CK = 65536


def _make_count(n_tokens: int, vocab_pad: int):
    per_w = n_tokens // _NW
    nch = per_w // _SCHUNK
    last_stripe = vocab_pad - (_NS - 1) * _STRIPE

    mesh = plsc.VectorSubcoreMesh(core_axis_name="c", subcore_axis_name="s")

    @functools.partial(
        pl.kernel,
        out_type=jax.ShapeDtypeStruct((_NC, vocab_pad), jnp.float32),
        mesh=mesh,
        scratch_types=[
            pltpu.VMEM((nch, _SCHUNK), jnp.int32),
            pltpu.VMEM((_SCHUNK,), jnp.float32),
            pltpu.VMEM_SHARED((vocab_pad,), jnp.float32),
            pltpu.SemaphoreType.DMA,
            pltpu.SemaphoreType.DMA,
        ],
    )
    def count(idx_hbm, zeros_hbm, out_hbm, idx_v, ones_v, cnt_sh, sem_i,
              sem_z):
        cid = lax.axis_index("c")
        sid = lax.axis_index("s")
        wid = sid * _NC + cid

        ci = pltpu.async_copy(idx_hbm.at[wid], idx_v, sem_i)

        for g in range(_SCHUNK // _LANES):
            ones_v[pl.ds(g * _LANES, _LANES)] = jnp.ones(
                (_LANES,), jnp.float32
            )

        base = sid * _STRIPE

        @pl.when(sid == _NS - 1)
        def _():
            pltpu.async_copy(
                zeros_hbm, cnt_sh.at[pl.ds(base, last_stripe)], sem_z
            ).wait()

        @pl.when(sid != _NS - 1)
        def _():
            pltpu.async_copy(
                zeros_hbm.at[pl.ds(0, _STRIPE)],
                cnt_sh.at[pl.ds(base, _STRIPE)],
                sem_z,
            ).wait()

        ci.wait()
        plsc.subcore_barrier()

        for j in range(nch):
            pltpu.sync_copy(ones_v, cnt_sh.at[idx_v.at[j]], add=True)

        plsc.subcore_barrier()

        row = out_hbm.at[cid]

        @pl.when(sid == _NS - 1)
        def _():
            pltpu.async_copy(
                cnt_sh.at[pl.ds(base, last_stripe)],
                row.at[pl.ds(base, last_stripe)],
                sem_z,
            ).wait()

        @pl.when(sid != _NS - 1)
        def _():
            pltpu.async_copy(
                cnt_sh.at[pl.ds(base, _STRIPE)],
                row.at[pl.ds(base, _STRIPE)],
                sem_z,
            ).wait()

    return count


def _matvec_head_kernel(nblk, vocab, cnt_ref, t_ref, w_ref, b_ref, o_ref,
                        acc_ref):
    k = pl.program_id(0)
    csum = cnt_ref[0:1, :] + cnt_ref[1:2, :]
    t = t_ref[...]

    def tail_psum(_):
        gid = (nblk - 1) * _CK + lax.broadcasted_iota(
            jnp.int32, (1, _CK), 1
        )
        prod = jnp.where(gid < vocab, t * csum, 0.0)
        return jnp.sum(prod, axis=1, keepdims=True)

    def main_psum(_):
        return jnp.sum(t * csum, axis=1, keepdims=True)

    psum = lax.cond(k == nblk - 1, tail_psum, main_psum, 0)

    @pl.when(k == 0)
    def _():
        acc_ref[...] = jnp.zeros_like(acc_ref)

    acc_ref[...] += psum

    @pl.when(k == nblk - 1)
    def _():
        bow = acc_ref[...]
        logits = (
            jnp.dot(w_ref[...], bow, preferred_element_type=jnp.float32)
            + b_ref[...]
        )
        m = jnp.max(logits, axis=0, keepdims=True)
        e = jnp.exp(logits - m)
        lse = jnp.log(jnp.sum(e, axis=0, keepdims=True)) + m
        o_ref[...] = logits - lse


def kernel(inputs, emb_table, W, b):
    n_tokens = inputs.shape[0]
    vocab = emb_table.shape[0]
    num_labels = W.shape[0]

    per_w = n_tokens // _NW
    idx = inputs.astype(jnp.int32).reshape(_NW, per_w // _SCHUNK, _SCHUNK)

    vocab_pad = -(-vocab // 128) * 128
    last_stripe = vocab_pad - (_NS - 1) * _STRIPE
    zeros = jnp.zeros((last_stripe,), jnp.float32)
    cnt2 = _make_count(n_tokens, vocab_pad)(idx, zeros)

    table_t = emb_table.T

    nblk = (vocab + _CK - 1) // _CK
    out_col = pl.pallas_call(
        functools.partial(_matvec_head_kernel, nblk, vocab),
        grid=(nblk,),
        in_specs=[
            pl.BlockSpec((_NC, _CK), lambda k: (0, k)),
            pl.BlockSpec((_D, _CK), lambda k: (0, k)),
            pl.BlockSpec((num_labels, _D), lambda k: (0, 0)),
            pl.BlockSpec((num_labels, 1), lambda k: (0, 0)),
        ],
        out_specs=pl.BlockSpec((num_labels, 1), lambda k: (0, 0)),
        out_shape=jax.ShapeDtypeStruct((num_labels, 1), jnp.float32),
        scratch_shapes=[pltpu.VMEM((_D, 1), jnp.float32)],
    )(cnt2, table_t, W, b.reshape(num_labels, 1))

    return out_col.reshape(1, num_labels)
